# baseline (speedup 1.0000x reference)
import functools

import jax
import jax.numpy as jnp
from jax import lax
from jax.experimental import pallas as pl
from jax.experimental.pallas import tpu as pltpu
from jax.experimental.pallas import tpu_sc as plsc

BATCH = 16384
EMB = 64
NROWS = 1000001
_TBLK = 16384
_TGRID = (NROWS + _TBLK - 1) // _TBLK
ZROWS = _TGRID * _TBLK

_info = plsc.get_sparse_core_info()
_NC, _NS = _info.num_cores, _info.num_subcores
_NW = _NC * _NS
_BPW = BATCH // _NW


def _transpose_body(wuT_ref, wiT_ref, eye_ref, z_ref):
    x = jnp.concatenate([wuT_ref[...], wiT_ref[...]], axis=0)
    del eye_ref
    z_ref[...] = x.T


def _tc_transpose(wuT, wiT):
    eye = jnp.eye(2 * EMB, dtype=jnp.bfloat16)
    return pl.pallas_call(
        _transpose_body,
        grid=(_TGRID,),
        in_specs=[
            pl.BlockSpec((EMB, _TBLK), lambda g: (0, g)),
            pl.BlockSpec((EMB, _TBLK), lambda g: (0, g)),
            pl.BlockSpec((2 * EMB, 2 * EMB), lambda g: (0, 0)),
        ],
        out_specs=pl.BlockSpec((_TBLK, 2 * EMB), lambda g: (g, 0)),
        out_shape=jax.ShapeDtypeStruct((ZROWS, 2 * EMB), jnp.float32),
        compiler_params=pltpu.CompilerParams(
            dimension_semantics=("arbitrary",)),
    )(wuT, wiT, eye)


def _gather_body(z_hbm, u_hbm, i_hbm, eu_hbm, ei_hbm, idx_v, dst, sem):
    wid = lax.axis_index("s") * _NC + lax.axis_index("c")
    base = wid * _BPW

    def do_table(idx_hbm, out_ref):
        pltpu.sync_copy(idx_hbm.at[pl.ds(base, _BPW)], idx_v)
        copies = [
            pltpu.async_copy(z_hbm.at[idx_v.at[pl.ds(q * 128, 128)]],
                             dst.at[pl.ds(q * 128, 128)], sem)
            for q in range(_BPW // 128)
        ]
        for cp in copies:
            cp.wait()
        pltpu.sync_copy(dst, out_ref.at[pl.ds(base, _BPW)])

    do_table(u_hbm, eu_hbm)
    do_table(i_hbm, ei_hbm)


def _sc_gather(z, u, i):
    mesh = plsc.VectorSubcoreMesh(core_axis_name="c", subcore_axis_name="s")
    f = functools.partial(
        pl.kernel,
        mesh=mesh,
        out_type=(
            jax.ShapeDtypeStruct((BATCH, 2 * EMB), jnp.float32),
            jax.ShapeDtypeStruct((BATCH, 2 * EMB), jnp.float32),
        ),
        scratch_types=[
            pltpu.VMEM((_BPW,), jnp.int32),
            pltpu.VMEM((_BPW, 2 * EMB), jnp.float32),
            pltpu.SemaphoreType.DMA,
        ],
    )(_gather_body)
    return f(z, u, i)


def _mlp_body(eu_ref, ei_ref, w1a_ref, w1b_ref, b1_ref, w2_ref, b2_ref,
              w3_ref, b3_ref, out_ref):
    h = jnp.dot(eu_ref[...], w1a_ref[...], preferred_element_type=jnp.float32)
    h = h + jnp.dot(ei_ref[...], w1b_ref[...],
                    preferred_element_type=jnp.float32)
    h = jnp.maximum(h + b1_ref[...], 0.0)
    h = jnp.maximum(
        jnp.dot(h, w2_ref[...], preferred_element_type=jnp.float32)
        + b2_ref[...], 0.0)
    o = jnp.dot(h, w3_ref[...], preferred_element_type=jnp.float32) + b3_ref[...]
    out_ref[...] = jax.nn.sigmoid(o)


def _tc_mlp(eu, ei, w1, b1, w2, b2, w3, b3):
    blk = 2048
    grid = BATCH // blk
    zpad = jnp.zeros((EMB, 128), jnp.float32)
    w1a = jnp.concatenate([w1[:EMB], zpad], axis=0)
    w1b = jnp.concatenate([zpad, w1[EMB:]], axis=0)
    full = lambda shape: pl.BlockSpec(shape, lambda g: (0, 0))
    out = pl.pallas_call(
        _mlp_body,
        grid=(grid,),
        in_specs=[
            pl.BlockSpec((blk, 2 * EMB), lambda g: (g, 0)),
            pl.BlockSpec((blk, 2 * EMB), lambda g: (g, 0)),
            full((128, 128)),
            full((128, 128)),
            full((1, 128)),
            full((128, 64)),
            full((1, 64)),
            full((64, 1)),
            full((1, 1)),
        ],
        out_specs=pl.BlockSpec((blk, 1), lambda g: (g, 0)),
        out_shape=jax.ShapeDtypeStruct((BATCH, 1), jnp.float32),
        compiler_params=pltpu.CompilerParams(
            dimension_semantics=("arbitrary",)),
    )(eu, ei, w1a, w1b, b1.reshape(1, 128), w2, b2.reshape(1, 64), w3,
      b3.reshape(1, 1))
    return jnp.squeeze(out, axis=-1)


def kernel(u, i, Wu, Wi, W1, b1, W2, b2, W3, b3):
    z = _tc_transpose(Wu.T, Wi.T)
    eu, ei = _sc_gather(z, u.astype(jnp.int32), i.astype(jnp.int32))
    return _tc_mlp(eu, ei, W1, b1, W2, b2, W3, b3)


# speedup vs baseline: 1.1259x; 1.1259x over previous
import functools

import jax
import jax.numpy as jnp
from jax import lax
from jax.experimental import pallas as pl
from jax.experimental.pallas import tpu as pltpu
from jax.experimental.pallas import tpu_sc as plsc

BATCH = 16384
EMB = 64
NROWS = 1000001
_TBLK = 16384
_TGRID = (NROWS + _TBLK - 1) // _TBLK
ZROWS = _TGRID * _TBLK

_info = plsc.get_sparse_core_info()
_NC, _NS = _info.num_cores, _info.num_subcores
_NW = _NC * _NS
_BPW = BATCH // _NW


def _transpose_body(wuT_ref, wiT_ref, eye_ref, z_ref):
    x = jnp.concatenate(
        [wuT_ref[...], wiT_ref[...]], axis=0).astype(jnp.bfloat16)
    dn = (((0,), (0,)), ((), ()))
    z_ref[...] = lax.dot_general(
        x, eye_ref[...], dn,
        preferred_element_type=jnp.float32).astype(jnp.bfloat16)


def _tc_transpose(wuT, wiT):
    eye = jnp.eye(2 * EMB, dtype=jnp.bfloat16)
    return pl.pallas_call(
        _transpose_body,
        grid=(_TGRID,),
        in_specs=[
            pl.BlockSpec((EMB, _TBLK), lambda g: (0, g)),
            pl.BlockSpec((EMB, _TBLK), lambda g: (0, g)),
            pl.BlockSpec((2 * EMB, 2 * EMB), lambda g: (0, 0)),
        ],
        out_specs=pl.BlockSpec((_TBLK, 2 * EMB), lambda g: (g, 0)),
        out_shape=jax.ShapeDtypeStruct((ZROWS, 2 * EMB), jnp.bfloat16),
        compiler_params=pltpu.CompilerParams(
            dimension_semantics=("arbitrary",)),
    )(wuT, wiT, eye)


def _gather_body(z_hbm, u_hbm, i_hbm, eu_hbm, ei_hbm,
                 idx_v, idx2_v, dst, stage, sem):
    wid = lax.axis_index("s") * _NC + lax.axis_index("c")
    base = wid * _BPW
    zf = z_hbm.bitcast(jnp.float32)

    def do_table(idx_hbm, out_ref):
        pltpu.sync_copy(idx_hbm.at[pl.ds(base, _BPW)], idx_v)
        for q in range(_BPW // 16):
            v = idx_v[pl.ds(q * 16, 16)]
            idx2_v[pl.ds(q * 16, 16)] = v >> 1
        copies = [
            pltpu.async_copy(zf.at[idx2_v.at[pl.ds(q * 128, 128)]],
                             dst.at[pl.ds(q * 128, 128)], sem)
            for q in range(_BPW // 128)
        ]
        for cp in copies:
            cp.wait()

        def ext(it, carry):
            par16 = ((idx_v[pl.ds(it * 16, 16)] & 1) * 16).astype(jnp.uint32)
            for kk in range(16):
                k = it * 16 + kk
                sh = lax.gather(
                    par16, jnp.full((16, 1), kk, jnp.int32),
                    lax.GatherDimensionNumbers(
                        offset_dims=(), collapsed_slice_dims=(0,),
                        start_index_map=(0,)),
                    (1,), mode=lax.GatherScatterMode.PROMISE_IN_BOUNDS)
                row = dst.at[k]
                for g in range(8):
                    w = plsc.bitcast(row[pl.ds(g * 16, 16)], jnp.uint32)
                    o = plsc.bitcast((w >> sh) << 16, jnp.float32)
                    stage[kk, pl.ds(g * 16, 16)] = o
            pltpu.sync_copy(stage, out_ref.at[pl.ds(base + it * 16, 16)])
            return carry

        lax.fori_loop(0, _BPW // 16, ext, 0)

    do_table(u_hbm, eu_hbm)
    do_table(i_hbm, ei_hbm)


def _sc_gather(z, u, i):
    mesh = plsc.VectorSubcoreMesh(core_axis_name="c", subcore_axis_name="s")
    f = functools.partial(
        pl.kernel,
        mesh=mesh,
        out_type=(
            jax.ShapeDtypeStruct((BATCH, 2 * EMB), jnp.float32),
            jax.ShapeDtypeStruct((BATCH, 2 * EMB), jnp.float32),
        ),
        scratch_types=[
            pltpu.VMEM((_BPW,), jnp.int32),
            pltpu.VMEM((_BPW,), jnp.int32),
            pltpu.VMEM((_BPW, 2 * EMB), jnp.float32),
            pltpu.VMEM((16, 2 * EMB), jnp.float32),
            pltpu.SemaphoreType.DMA,
        ],
        compiler_params=pltpu.CompilerParams(
            use_tc_tiling_on_sc=True, needs_layout_passes=False),
    )(_gather_body)
    return f(z, u, i)


def _mlp_body(eu_ref, ei_ref, w1a_ref, w1b_ref, b1_ref, w2_ref, b2_ref,
              w3_ref, b3_ref, out_ref):
    h = jnp.dot(eu_ref[...], w1a_ref[...], preferred_element_type=jnp.float32)
    h = h + jnp.dot(ei_ref[...], w1b_ref[...],
                    preferred_element_type=jnp.float32)
    h = jnp.maximum(h + b1_ref[...], 0.0)
    h = jnp.maximum(
        jnp.dot(h, w2_ref[...], preferred_element_type=jnp.float32)
        + b2_ref[...], 0.0)
    o = jnp.dot(h, w3_ref[...], preferred_element_type=jnp.float32) + b3_ref[...]
    out_ref[...] = jax.nn.sigmoid(o)


def _tc_mlp(eu, ei, w1, b1, w2, b2, w3, b3):
    blk = 2048
    grid = BATCH // blk
    zpad = jnp.zeros((EMB, 128), jnp.float32)
    w1a = jnp.concatenate([w1[:EMB], zpad], axis=0)
    w1b = jnp.concatenate([zpad, w1[EMB:]], axis=0)
    full = lambda shape: pl.BlockSpec(shape, lambda g: (0, 0))
    out = pl.pallas_call(
        _mlp_body,
        grid=(grid,),
        in_specs=[
            pl.BlockSpec((blk, 2 * EMB), lambda g: (g, 0)),
            pl.BlockSpec((blk, 2 * EMB), lambda g: (g, 0)),
            full((128, 128)),
            full((128, 128)),
            full((1, 128)),
            full((128, 64)),
            full((1, 64)),
            full((64, 1)),
            full((1, 1)),
        ],
        out_specs=pl.BlockSpec((blk, 1), lambda g: (g, 0)),
        out_shape=jax.ShapeDtypeStruct((BATCH, 1), jnp.float32),
        compiler_params=pltpu.CompilerParams(
            dimension_semantics=("arbitrary",)),
    )(eu, ei, w1a, w1b, b1.reshape(1, 128), w2, b2.reshape(1, 64), w3,
      b3.reshape(1, 1))
    return jnp.squeeze(out, axis=-1)


def kernel(u, i, Wu, Wi, W1, b1, W2, b2, W3, b3):
    z = _tc_transpose(Wu.T, Wi.T)
    eu, ei = _sc_gather(z, u.astype(jnp.int32), i.astype(jnp.int32))
    return _tc_mlp(eu, ei, W1, b1, W2, b2, W3, b3)


# speedup vs baseline: 1.1318x; 1.0052x over previous
import functools

import jax
import jax.numpy as jnp
from jax import lax
from jax.experimental import pallas as pl
from jax.experimental.pallas import tpu as pltpu
from jax.experimental.pallas import tpu_sc as plsc

BATCH = 16384
EMB = 64
NROWS = 1000001
_TBLK = 16384
_TGRID = (NROWS + _TBLK - 1) // _TBLK
ZROWS = _TGRID * _TBLK

_info = plsc.get_sparse_core_info()
_NC, _NS = _info.num_cores, _info.num_subcores
_NW = _NC * _NS
_BPW = BATCH // _NW


def _transpose_body(wuT_ref, wiT_ref, eye_ref, z_ref):
    x = jnp.concatenate(
        [wuT_ref[...], wiT_ref[...]], axis=0).astype(jnp.bfloat16)
    dn = (((0,), (0,)), ((), ()))
    z_ref[...] = lax.dot_general(
        x, eye_ref[...], dn,
        preferred_element_type=jnp.float32).astype(jnp.bfloat16)


def _tc_transpose(wuT, wiT):
    eye = jnp.eye(2 * EMB, dtype=jnp.bfloat16)
    return pl.pallas_call(
        _transpose_body,
        grid=(_TGRID,),
        in_specs=[
            pl.BlockSpec((EMB, _TBLK), lambda g: (0, g)),
            pl.BlockSpec((EMB, _TBLK), lambda g: (0, g)),
            pl.BlockSpec((2 * EMB, 2 * EMB), lambda g: (0, 0)),
        ],
        out_specs=pl.BlockSpec((_TBLK, 2 * EMB), lambda g: (g, 0)),
        out_shape=jax.ShapeDtypeStruct((ZROWS, 2 * EMB), jnp.bfloat16),
        compiler_params=pltpu.CompilerParams(
            dimension_semantics=("arbitrary",)),
    )(wuT, wiT, eye)


def _gather_body(z_hbm, u_hbm, i_hbm, eu_hbm, ei_hbm,
                 idx_v, idx2_v, dst, stage, sem):
    wid = lax.axis_index("s") * _NC + lax.axis_index("c")
    base = wid * _BPW
    zf = z_hbm.bitcast(jnp.float32)

    def do_table(idx_hbm, out_ref):
        pltpu.sync_copy(idx_hbm.at[pl.ds(base, _BPW)], idx_v)
        for q in range(_BPW // 16):
            v = idx_v[pl.ds(q * 16, 16)]
            idx2_v[pl.ds(q * 16, 16)] = v >> 1
        copies = [
            pltpu.async_copy(zf.at[idx2_v.at[pl.ds(q * 128, 128)]],
                             dst.at[pl.ds(q * 128, 128)], sem)
            for q in range(_BPW // 128)
        ]
        for cp in copies:
            cp.wait()

        def ext(it, carry):
            par16 = ((idx_v[pl.ds(it * 16, 16)] & 1) * 16).astype(jnp.uint32)
            for kk in range(16):
                k = it * 16 + kk
                sh = lax.gather(
                    par16, jnp.full((16, 1), kk, jnp.int32),
                    lax.GatherDimensionNumbers(
                        offset_dims=(), collapsed_slice_dims=(0,),
                        start_index_map=(0,)),
                    (1,), mode=lax.GatherScatterMode.PROMISE_IN_BOUNDS)
                row = dst.at[k]
                for g in range(8):
                    w = plsc.bitcast(row[pl.ds(g * 16, 16)], jnp.uint32)
                    o = plsc.bitcast((w >> sh) << 16, jnp.float32)
                    stage[kk, pl.ds(g * 16, 16)] = o
            pltpu.sync_copy(stage, out_ref.at[pl.ds(base + it * 16, 16)])
            return carry

        lax.fori_loop(0, _BPW // 16, ext, 0)

    do_table(u_hbm, eu_hbm)
    do_table(i_hbm, ei_hbm)


def _sc_gather(z, u, i):
    mesh = plsc.VectorSubcoreMesh(core_axis_name="c", subcore_axis_name="s")
    f = functools.partial(
        pl.kernel,
        mesh=mesh,
        out_type=(
            jax.ShapeDtypeStruct((BATCH, 2 * EMB), jnp.float32),
            jax.ShapeDtypeStruct((BATCH, 2 * EMB), jnp.float32),
        ),
        scratch_types=[
            pltpu.VMEM((_BPW,), jnp.int32),
            pltpu.VMEM((_BPW,), jnp.int32),
            pltpu.VMEM((_BPW, 2 * EMB), jnp.float32),
            pltpu.VMEM((16, 2 * EMB), jnp.float32),
            pltpu.SemaphoreType.DMA,
        ],
        compiler_params=pltpu.CompilerParams(
            use_tc_tiling_on_sc=True, needs_layout_passes=False),
    )(_gather_body)
    return f(z, u, i)


def _mlp_body(eu_ref, ei_ref, w1a_ref, w1b_ref, b1_ref, w2_ref, b2_ref,
              w3_ref, b3_ref, out_ref):
    eu = eu_ref[...].astype(jnp.bfloat16)
    ei = ei_ref[...].astype(jnp.bfloat16)
    h = jnp.dot(eu, w1a_ref[...], preferred_element_type=jnp.float32)
    h = h + jnp.dot(ei, w1b_ref[...],
                    preferred_element_type=jnp.float32)
    h = jnp.maximum(h + b1_ref[...], 0.0)
    h = jnp.maximum(
        jnp.dot(h, w2_ref[...], preferred_element_type=jnp.float32)
        + b2_ref[...], 0.0)
    o = jnp.dot(h, w3_ref[...], preferred_element_type=jnp.float32) + b3_ref[...]
    out_ref[...] = jax.nn.sigmoid(o)


def _tc_mlp(eu, ei, w1, b1, w2, b2, w3, b3):
    blk = 8192
    grid = BATCH // blk
    zpad = jnp.zeros((EMB, 128), jnp.float32)
    w1a = jnp.concatenate([w1[:EMB], zpad], axis=0).astype(jnp.bfloat16)
    w1b = jnp.concatenate([zpad, w1[EMB:]], axis=0).astype(jnp.bfloat16)
    full = lambda shape: pl.BlockSpec(shape, lambda g: (0, 0))
    out = pl.pallas_call(
        _mlp_body,
        grid=(grid,),
        in_specs=[
            pl.BlockSpec((blk, 2 * EMB), lambda g: (g, 0)),
            pl.BlockSpec((blk, 2 * EMB), lambda g: (g, 0)),
            full((128, 128)),
            full((128, 128)),
            full((1, 128)),
            full((128, 64)),
            full((1, 64)),
            full((64, 1)),
            full((1, 1)),
        ],
        out_specs=pl.BlockSpec((blk, 1), lambda g: (g, 0)),
        out_shape=jax.ShapeDtypeStruct((BATCH, 1), jnp.float32),
        compiler_params=pltpu.CompilerParams(
            dimension_semantics=("arbitrary",)),
    )(eu, ei, w1a, w1b, b1.reshape(1, 128), w2, b2.reshape(1, 64), w3,
      b3.reshape(1, 1))
    return jnp.squeeze(out, axis=-1)


def kernel(u, i, Wu, Wi, W1, b1, W2, b2, W3, b3):
    z = _tc_transpose(Wu.T, Wi.T)
    eu, ei = _sc_gather(z, u.astype(jnp.int32), i.astype(jnp.int32))
    return _tc_mlp(eu, ei, W1, b1, W2, b2, W3, b3)


# speedup vs baseline: 1.1395x; 1.0068x over previous
import functools

import jax
import jax.numpy as jnp
from jax import lax
from jax.experimental import pallas as pl
from jax.experimental.pallas import tpu as pltpu
from jax.experimental.pallas import tpu_sc as plsc

BATCH = 16384
EMB = 64
NROWS = 1000001
_TBLK = 16384
_TGRID = (NROWS + _TBLK - 1) // _TBLK
ZROWS = _TGRID * _TBLK

_info = plsc.get_sparse_core_info()
_NC, _NS = _info.num_cores, _info.num_subcores
_NW = _NC * _NS
_BPW = BATCH // _NW


def _transpose_body(wuT_ref, wiT_ref, eye_ref, z_ref):
    x = jnp.concatenate(
        [wuT_ref[...], wiT_ref[...]], axis=0).astype(jnp.bfloat16)
    dn = (((0,), (0,)), ((), ()))
    z_ref[...] = lax.dot_general(
        x, eye_ref[...], dn,
        preferred_element_type=jnp.float32).astype(jnp.bfloat16)


def _tc_transpose(wuT, wiT):
    eye = jnp.eye(2 * EMB, dtype=jnp.bfloat16)
    return pl.pallas_call(
        _transpose_body,
        grid=(_TGRID,),
        in_specs=[
            pl.BlockSpec((EMB, _TBLK), lambda g: (0, g)),
            pl.BlockSpec((EMB, _TBLK), lambda g: (0, g)),
            pl.BlockSpec((2 * EMB, 2 * EMB), lambda g: (0, 0)),
        ],
        out_specs=pl.BlockSpec((_TBLK, 2 * EMB), lambda g: (g, 0)),
        out_shape=jax.ShapeDtypeStruct((ZROWS, 2 * EMB), jnp.bfloat16),
        compiler_params=pltpu.CompilerParams(
            dimension_semantics=("arbitrary",)),
    )(wuT, wiT, eye)


_HCH = _BPW // 2


def _gather_body(z_hbm, u_hbm, i_hbm, eu_hbm, ei_hbm,
                 idxu_v, idxi_v, idx2u_v, idx2i_v, dstA, dstB, stage,
                 semA, semB):
    wid = lax.axis_index("s") * _NC + lax.axis_index("c")
    base = wid * _BPW
    zf = z_hbm.bitcast(jnp.float32)

    def stage_idx(idx_hbm, iv, i2v):
        pltpu.sync_copy(idx_hbm.at[pl.ds(base, _BPW)], iv)
        for q in range(_BPW // 16):
            i2v[pl.ds(q * 16, 16)] = iv[pl.ds(q * 16, 16)] >> 1

    def fire(i2v, off, dbuf, sem):
        return [
            pltpu.async_copy(zf.at[i2v.at[pl.ds(off + j * 128, 128)]],
                             dbuf.at[pl.ds(j * 128, 128)], sem)
            for j in range(_HCH // 128)
        ]

    def extract(copies, dbuf, iv, off, out_ref):
        for cp in copies:
            cp.wait()

        def ext(it, carry):
            par16 = ((iv[pl.ds(off + it * 16, 16)] & 1) * 16).astype(
                jnp.uint32)
            for kk in range(16):
                sh = lax.gather(
                    par16, jnp.full((16, 1), kk, jnp.int32),
                    lax.GatherDimensionNumbers(
                        offset_dims=(), collapsed_slice_dims=(0,),
                        start_index_map=(0,)),
                    (1,), mode=lax.GatherScatterMode.PROMISE_IN_BOUNDS)
                row = dbuf.at[it * 16 + kk]
                for g in range(8):
                    w = plsc.bitcast(row[pl.ds(g * 16, 16)], jnp.uint32)
                    o = plsc.bitcast((w >> sh) << 16, jnp.float32)
                    stage[kk, pl.ds(g * 16, 16)] = o
            pltpu.sync_copy(stage,
                            out_ref.at[pl.ds(base + off + it * 16, 16)])
            return carry

        lax.fori_loop(0, _HCH // 16, ext, 0)

    stage_idx(u_hbm, idxu_v, idx2u_v)
    cA = fire(idx2u_v, 0, dstA, semA)
    cB = fire(idx2u_v, _HCH, dstB, semB)
    stage_idx(i_hbm, idxi_v, idx2i_v)
    extract(cA, dstA, idxu_v, 0, eu_hbm)
    cA2 = fire(idx2i_v, 0, dstA, semA)
    extract(cB, dstB, idxu_v, _HCH, eu_hbm)
    cB2 = fire(idx2i_v, _HCH, dstB, semB)
    extract(cA2, dstA, idxi_v, 0, ei_hbm)
    extract(cB2, dstB, idxi_v, _HCH, ei_hbm)


def _sc_gather(z, u, i):
    mesh = plsc.VectorSubcoreMesh(core_axis_name="c", subcore_axis_name="s")
    f = functools.partial(
        pl.kernel,
        mesh=mesh,
        out_type=(
            jax.ShapeDtypeStruct((BATCH, 2 * EMB), jnp.float32),
            jax.ShapeDtypeStruct((BATCH, 2 * EMB), jnp.float32),
        ),
        scratch_types=[
            pltpu.VMEM((_BPW,), jnp.int32),
            pltpu.VMEM((_BPW,), jnp.int32),
            pltpu.VMEM((_BPW,), jnp.int32),
            pltpu.VMEM((_BPW,), jnp.int32),
            pltpu.VMEM((_HCH, 2 * EMB), jnp.float32),
            pltpu.VMEM((_HCH, 2 * EMB), jnp.float32),
            pltpu.VMEM((16, 2 * EMB), jnp.float32),
            pltpu.SemaphoreType.DMA,
            pltpu.SemaphoreType.DMA,
        ],
        compiler_params=pltpu.CompilerParams(
            use_tc_tiling_on_sc=True, needs_layout_passes=False),
    )(_gather_body)
    return f(z, u, i)


def _mlp_body(eu_ref, ei_ref, w1a_ref, w1b_ref, b1_ref, w2_ref, b2_ref,
              w3_ref, b3_ref, out_ref):
    eu = eu_ref[...].astype(jnp.bfloat16)
    ei = ei_ref[...].astype(jnp.bfloat16)
    h = jnp.dot(eu, w1a_ref[...], preferred_element_type=jnp.float32)
    h = h + jnp.dot(ei, w1b_ref[...],
                    preferred_element_type=jnp.float32)
    h = jnp.maximum(h + b1_ref[...], 0.0)
    h = jnp.maximum(
        jnp.dot(h, w2_ref[...], preferred_element_type=jnp.float32)
        + b2_ref[...], 0.0)
    o = jnp.dot(h, w3_ref[...], preferred_element_type=jnp.float32) + b3_ref[...]
    out_ref[...] = jax.nn.sigmoid(o)


def _tc_mlp(eu, ei, w1, b1, w2, b2, w3, b3):
    blk = 8192
    grid = BATCH // blk
    zpad = jnp.zeros((EMB, 128), jnp.float32)
    w1a = jnp.concatenate([w1[:EMB], zpad], axis=0).astype(jnp.bfloat16)
    w1b = jnp.concatenate([zpad, w1[EMB:]], axis=0).astype(jnp.bfloat16)
    full = lambda shape: pl.BlockSpec(shape, lambda g: (0, 0))
    out = pl.pallas_call(
        _mlp_body,
        grid=(grid,),
        in_specs=[
            pl.BlockSpec((blk, 2 * EMB), lambda g: (g, 0)),
            pl.BlockSpec((blk, 2 * EMB), lambda g: (g, 0)),
            full((128, 128)),
            full((128, 128)),
            full((1, 128)),
            full((128, 64)),
            full((1, 64)),
            full((64, 1)),
            full((1, 1)),
        ],
        out_specs=pl.BlockSpec((blk, 1), lambda g: (g, 0)),
        out_shape=jax.ShapeDtypeStruct((BATCH, 1), jnp.float32),
        compiler_params=pltpu.CompilerParams(
            dimension_semantics=("arbitrary",)),
    )(eu, ei, w1a, w1b, b1.reshape(1, 128), w2, b2.reshape(1, 64), w3,
      b3.reshape(1, 1))
    return jnp.squeeze(out, axis=-1)


def kernel(u, i, Wu, Wi, W1, b1, W2, b2, W3, b3):
    z = _tc_transpose(Wu.T, Wi.T)
    eu, ei = _sc_gather(z, u.astype(jnp.int32), i.astype(jnp.int32))
    return _tc_mlp(eu, ei, W1, b1, W2, b2, W3, b3)
